# Initial kernel scaffold; baseline (speedup 1.0000x reference)
import jax
import jax.numpy as jnp
from jax.experimental import pallas as pl


def kernel(x, edge_index, edge_attr, W_fc, W_edge, W_att):
    raise NotImplementedError("write your pallas kernel here")


# speedup vs baseline: 7.6717x; 7.6717x over previous
import functools

import jax
import jax.numpy as jnp
from jax import lax
from jax.experimental import pallas as pl
from jax.experimental.pallas import tpu as pltpu
from jax.experimental.pallas import tpu_sc as plsc

N_NODES = 10000
N_EDGES = 320000
CH = 128

NC = 2
NS = 16
NW = NC * NS
EPW = N_EDGES // NW
K = 80
CHUNKS = EPW // K
N_PAD = 10240
RPT = N_PAD // NS


def _sc_aggregate(x, row, col, zeros):
    mesh = plsc.VectorSubcoreMesh(core_axis_name="c", subcore_axis_name="s")

    @functools.partial(
        pl.kernel,
        mesh=mesh,
        out_type=jax.ShapeDtypeStruct((NC, N_PAD, CH), jnp.float32),
        scratch_types=[
            pltpu.VMEM((K,), jnp.int32),
            pltpu.VMEM((K,), jnp.int32),
            pltpu.VMEM((K, CH), jnp.float32),
            pltpu.VMEM_SHARED((N_PAD, CH), jnp.float32),
            pltpu.SemaphoreType.DMA,
        ],
    )
    def agg_kernel(x_hbm, row_hbm, col_hbm, z_hbm, out_hbm,
                   colv, rowv, gbuf, acc, sem):
        c = lax.axis_index("c")
        s = lax.axis_index("s")
        wid = c * NS + s
        pltpu.sync_copy(z_hbm.at[pl.ds(s * RPT, RPT)],
                        acc.at[pl.ds(s * RPT, RPT)])
        plsc.subcore_barrier()

        base = wid * EPW

        def body(k, carry):
            off = base + k * K
            pltpu.sync_copy(col_hbm.at[pl.ds(off, K)], colv)
            pltpu.sync_copy(row_hbm.at[pl.ds(off, K)], rowv)
            pltpu.async_copy(x_hbm.at[colv], gbuf, sem).wait()
            pltpu.sync_copy(gbuf, acc.at[rowv], add=True)
            return carry

        lax.fori_loop(0, CHUNKS, body, 0)
        plsc.subcore_barrier()
        pltpu.sync_copy(acc.at[pl.ds(s * RPT, RPT)],
                        out_hbm.at[c, pl.ds(s * RPT, RPT)])

    return agg_kernel(x, row, col, zeros)


ROWS_BLK = 2000


def _mm_body(p_ref, w_ref, o_ref):
    acc = p_ref[0] + p_ref[1]
    o_ref[...] = lax.dot_general(
        acc, w_ref[...], (((1,), (1,)), ((), ())),
        preferred_element_type=jnp.float32)


def _tc_matmul(partials, W_fc):
    return pl.pallas_call(
        _mm_body,
        grid=(N_NODES // ROWS_BLK,),
        in_specs=[
            pl.BlockSpec((NC, ROWS_BLK, CH), lambda i: (0, i, 0)),
            pl.BlockSpec((CH, CH), lambda i: (0, 0)),
        ],
        out_specs=pl.BlockSpec((ROWS_BLK, CH), lambda i: (i, 0)),
        out_shape=jax.ShapeDtypeStruct((N_NODES, CH), jnp.float32),
    )(partials, W_fc)


def kernel(x, edge_index, edge_attr, W_fc, W_edge, W_att):
    del edge_attr, W_edge, W_att
    ei = edge_index.astype(jnp.int32)
    row = ei[0]
    col = ei[1]
    zeros = jnp.zeros((N_PAD, CH), jnp.float32)
    partials = _sc_aggregate(x, row, col, zeros)
    return _tc_matmul(partials[:, :N_NODES, :], W_fc)
